# baseline (speedup 1.0000x reference)
import functools

import jax
import jax.numpy as jnp
from jax import lax
from jax.experimental import pallas as pl
from jax.experimental.pallas import tpu as pltpu
from jax.experimental.pallas import tpu_sc as plsc

N = 10000
E = 320000
D = 128
K = 64
TEMP = 30.0

NC = 2
NS = 16
NW = NC * NS
NPAD = 10240
ROWS_PER_TILE = NPAD // NS

CH = 128
NCHUNKS = E // CH
BASE_CHUNKS = NCHUNKS // NS
EXTRA = NCHUNKS - BASE_CHUNKS * NS
SS = 3
NSS = BASE_CHUNKS // SS
NPAIR = NSS // 2

DEGW = 16
DH = D // NC

BN = 1000
NB = N // BN


def _sc_edge_agg(xcat, src, dst):
    mesh = plsc.VectorSubcoreMesh(core_axis_name="c", subcore_axis_name="s")

    @functools.partial(
        pl.kernel,
        mesh=mesh,
        out_type=[
            jax.ShapeDtypeStruct((NC, NPAD, DH), jnp.float32),
            jax.ShapeDtypeStruct((NC, NPAD, DEGW), jnp.float32),
        ],
        scratch_types=[
            pltpu.VMEM((2, SS * CH), jnp.int32),
            pltpu.VMEM((2, SS, CH), jnp.int32),
            pltpu.VMEM((2, SS, CH, DH), jnp.float32),
            pltpu.VMEM((CH, DEGW), jnp.float32),
            pltpu.VMEM((CH, DEGW), jnp.float32),
            pltpu.VMEM_SHARED((NPAD, DH), jnp.float32),
            pltpu.VMEM_SHARED((NPAD, DEGW), jnp.float32),
            pltpu.SemaphoreType.DMA,
            pltpu.SemaphoreType.DMA,
            pltpu.SemaphoreType.DMA,
            pltpu.SemaphoreType.DMA,
        ],
        compiler_params=pltpu.CompilerParams(use_tc_tiling_on_sc=False),
    )
    def body(x_hbm, src_hbm, dst_hbm, agg_out, deg_out,
             sidx, didx, rows, halves, dzero, acc, dacc,
             gsem0, gsem1, ssem0, ssem1):
        gsem = (gsem0, gsem1)
        ssem = (ssem0, ssem1)
        cid = lax.axis_index("c")
        sid = lax.axis_index("s")
        zero16 = jnp.zeros((16,), jnp.float32)
        half16 = jnp.full((16,), 0.5, jnp.float32)

        def zrow(i, carry):
            for j in range(DH // 16):
                rows[0, 0, i, pl.ds(j * 16, 16)] = zero16
            return carry
        lax.fori_loop(0, CH, zrow, 0)

        def zdeg(i, carry):
            dzero[i, pl.ds(0, 16)] = zero16
            return carry
        lax.fori_loop(0, CH, zdeg, 0)

        def orow(i, carry):
            halves[i, pl.ds(0, 16)] = half16
            return carry
        lax.fori_loop(0, CH, orow, 0)

        row0 = sid * ROWS_PER_TILE
        for kblk in range(ROWS_PER_TILE // CH):
            pltpu.sync_copy(rows.at[0, 0],
                            acc.at[pl.ds(row0 + kblk * CH, CH), :])
            pltpu.sync_copy(dzero, dacc.at[pl.ds(row0 + kblk * CH, CH), :])
        plsc.subcore_barrier()

        start = sid * BASE_CHUNKS + jnp.minimum(sid, EXTRA)
        sbias = cid * N

        def fire_gathers(g, b):
            base = pl.multiple_of((start + g * SS) * CH, CH)
            pltpu.sync_copy(src_hbm.at[pl.ds(base, SS * CH)], sidx.at[b])
            for r in range(SS):
                pltpu.sync_copy(dst_hbm.at[pl.ds(base + r * CH, CH)],
                                didx.at[b, r])
            for j in range(SS * CH // 16):
                sidx[b, pl.ds(j * 16, 16)] = (
                    sidx[b, pl.ds(j * 16, 16)] + sbias)
            for r in range(SS):
                pltpu.async_copy(x_hbm.at[sidx.at[b, pl.ds(r * CH, CH)]],
                                 rows.at[b, r], gsem[b])

        def drain_gathers(b):
            for r in range(SS):
                pltpu.make_async_copy(x_hbm.at[sidx.at[b, pl.ds(r * CH, CH)]],
                                      rows.at[b, r], gsem[b]).wait()

        def fire_scatters(b):
            for r in range(SS):
                pltpu.async_copy(rows.at[b, r], acc.at[didx.at[b, r]],
                                 ssem[b], add=True)
                pltpu.async_copy(halves, dacc.at[didx.at[b, r]],
                                 ssem[b], add=True)

        def drain_scatters(b):
            for r in range(SS):
                pltpu.make_async_copy(rows.at[b, r], acc.at[didx.at[b, r]],
                                      ssem[b]).wait()
                pltpu.make_async_copy(halves, dacc.at[didx.at[b, r]],
                                      ssem[b]).wait()

        fire_gathers(0, 0)

        def pair(i, carry):
            fire_gathers(2 * i + 1, 1)
            drain_gathers(0)
            fire_scatters(0)
            drain_scatters(0)

            @pl.when(i < NPAIR - 1)
            def _():
                fire_gathers(2 * i + 2, 0)
            drain_gathers(1)
            fire_scatters(1)
            drain_scatters(1)
            return carry
        lax.fori_loop(0, NPAIR, pair, 0)

        @pl.when(sid < EXTRA)
        def _tail():
            base = pl.multiple_of((start + BASE_CHUNKS) * CH, CH)
            pltpu.sync_copy(src_hbm.at[pl.ds(base, CH)],
                            sidx.at[0, pl.ds(0, CH)])
            pltpu.sync_copy(dst_hbm.at[pl.ds(base, CH)], didx.at[0, 0])
            for j in range(CH // 16):
                sidx[0, pl.ds(j * 16, 16)] = (
                    sidx[0, pl.ds(j * 16, 16)] + sbias)
            pltpu.async_copy(x_hbm.at[sidx.at[0, pl.ds(0, CH)]],
                             rows.at[0, 0], gsem[0]).wait()
            pltpu.sync_copy(rows.at[0, 0], acc.at[didx.at[0, 0]], add=True)
            pltpu.sync_copy(halves, dacc.at[didx.at[0, 0]], add=True)

        plsc.subcore_barrier()

        pltpu.sync_copy(acc.at[pl.ds(row0, ROWS_PER_TILE), :],
                        agg_out.at[cid, pl.ds(row0, ROWS_PER_TILE), :])
        pltpu.sync_copy(dacc.at[pl.ds(row0, ROWS_PER_TILE), :],
                        deg_out.at[cid, pl.ds(row0, ROWS_PER_TILE), :])

    return body(xcat, src, dst)


def _phase_a_body(agg_ref, deg_ref, w_ref, cid_ref, z_ref, mu_ref, sums, cnts):
    i = pl.program_id(0)

    @pl.when(i == 0)
    def _init():
        sums[...] = jnp.zeros_like(sums)
        cnts[...] = jnp.zeros_like(cnts)

    h = (lax.dot_general(agg_ref[0], w_ref[0:DH, :], (((1,), (0,)), ((), ())),
                         preferred_element_type=jnp.float32)
         + lax.dot_general(agg_ref[1], w_ref[DH:D, :], (((1,), (0,)), ((), ())),
                           preferred_element_type=jnp.float32))
    dg = deg_ref[0, :, 0:1] + deg_ref[1, :, 0:1]
    h = h / jnp.maximum(dg, 1.0)
    h = jnp.maximum(h, 0.0)
    nrm = jnp.sqrt(jnp.sum(h * h, axis=1, keepdims=True))
    z = h / jnp.maximum(nrm, 1e-12)
    z_ref[...] = z

    cid = cid_ref[0]
    oht = (cid == lax.broadcasted_iota(jnp.int32, (K, 1), 0)
           ).astype(jnp.float32)
    sums[...] += lax.dot_general(oht, z, (((1,), (0,)), ((), ())),
                                 preferred_element_type=jnp.float32)
    cnts[...] += jnp.sum(oht, axis=1, keepdims=True)

    @pl.when(i == NB - 1)
    def _fin():
        mu_ref[...] = sums[...] / jnp.maximum(cnts[...], 1.0)


def _phase_a(agg, deg2, w, cids2):
    return pl.pallas_call(
        _phase_a_body,
        grid=(NB,),
        in_specs=[
            pl.BlockSpec((NC, BN, DH), lambda i: (0, i, 0)),
            pl.BlockSpec((NC, BN, DEGW), lambda i: (0, i, 0)),
            pl.BlockSpec((D, D), lambda i: (0, 0)),
            pl.BlockSpec((1, 1, BN), lambda i: (i, 0, 0)),
        ],
        out_specs=[
            pl.BlockSpec((BN, D), lambda i: (i, 0)),
            pl.BlockSpec((K, D), lambda i: (0, 0)),
        ],
        out_shape=[
            jax.ShapeDtypeStruct((N, D), jnp.float32),
            jax.ShapeDtypeStruct((K, D), jnp.float32),
        ],
        scratch_shapes=[
            pltpu.VMEM((K, D), jnp.float32),
            pltpu.VMEM((K, 1), jnp.float32),
        ],
    )(agg, deg2, w, cids2)


def _phase_b_body(z_ref, mu_ref, dist_ref, r_ref):
    z = z_ref[...]
    mu = mu_ref[...]
    d = lax.dot_general(z, mu, (((1,), (1,)), ((), ())),
                        preferred_element_type=jnp.float32)
    dist_ref[...] = d
    t = TEMP * d
    m = jnp.max(t, axis=1, keepdims=True)
    e = jnp.exp(t - m)
    r_ref[...] = e / jnp.sum(e, axis=1, keepdims=True)


def _phase_b(z_pad, mu):
    return pl.pallas_call(
        _phase_b_body,
        grid=(NB,),
        in_specs=[
            pl.BlockSpec((BN, D), lambda i: (i, 0)),
            pl.BlockSpec((K, D), lambda i: (0, 0)),
        ],
        out_specs=[
            pl.BlockSpec((BN, K), lambda i: (i, 0)),
            pl.BlockSpec((BN, K), lambda i: (i, 0)),
        ],
        out_shape=[
            jax.ShapeDtypeStruct((N, K), jnp.float32),
            jax.ShapeDtypeStruct((N, K), jnp.float32),
        ],
    )(z_pad, mu)


def kernel(x, W_enc, edge_index, community_ids):
    src = edge_index[0]
    dst = edge_index[1]
    xcat = jnp.concatenate([x[:, :DH], x[:, DH:]], axis=0)
    agg2, deg2 = _sc_edge_agg(xcat, src, dst)
    cids2 = community_ids.reshape(NB, 1, BN)
    z, mu = _phase_a(agg2, deg2, W_enc, cids2)
    dist, r = _phase_b(z, mu)
    return (z, mu, r, dist)


# speedup vs baseline: 1.2754x; 1.2754x over previous
import functools

import jax
import jax.numpy as jnp
from jax import lax
from jax.experimental import pallas as pl
from jax.experimental.pallas import tpu as pltpu
from jax.experimental.pallas import tpu_sc as plsc

N = 10000
E = 320000
D = 128
K = 64
TEMP = 30.0

NC = 2
NS = 16
NW = NC * NS
NPAD = 10240
ROWS_PER_TILE = NPAD // NS

CH = 128
NCHUNKS = E // CH
BASE_CHUNKS = NCHUNKS // NS
EXTRA = NCHUNKS - BASE_CHUNKS * NS
SS = 3
NSS = BASE_CHUNKS // SS
NPAIR = NSS // 2

DEGW = 16
DH = D // NC

BN = 1000
NB = N // BN


def _sc_edge_agg(xcat, src, dst):
    mesh = plsc.VectorSubcoreMesh(core_axis_name="c", subcore_axis_name="s")

    @functools.partial(
        pl.kernel,
        mesh=mesh,
        out_type=[
            jax.ShapeDtypeStruct((NC, NPAD, DH), jnp.float32),
            jax.ShapeDtypeStruct((NC, NPAD, DEGW), jnp.float32),
        ],
        scratch_types=[
            pltpu.VMEM((2, SS, CH), jnp.int32),
            pltpu.VMEM((2, SS, CH), jnp.int32),
            pltpu.VMEM((2, SS, CH, DH), jnp.float32),
            pltpu.VMEM((CH, DEGW), jnp.float32),
            pltpu.VMEM((CH, DEGW), jnp.float32),
            pltpu.VMEM_SHARED((NPAD, DH), jnp.float32),
            pltpu.VMEM_SHARED((NPAD, DEGW), jnp.float32),
            pltpu.SemaphoreType.DMA,
            pltpu.SemaphoreType.DMA,
            pltpu.SemaphoreType.DMA,
            pltpu.SemaphoreType.DMA,
            pltpu.SemaphoreType.DMA,
            pltpu.SemaphoreType.DMA,
        ],
        compiler_params=pltpu.CompilerParams(use_tc_tiling_on_sc=False),
    )
    def body(x_hbm, src_hbm, dst_hbm, agg_out, deg_out,
             sidx, didx, rows, halves, dzero, acc, dacc,
             gsem0, gsem1, ssem0, ssem1, isem0, isem1):
        gsem = (gsem0, gsem1)
        ssem = (ssem0, ssem1)
        isem = (isem0, isem1)
        cid = lax.axis_index("c")
        sid = lax.axis_index("s")
        zero16 = jnp.zeros((16,), jnp.float32)
        half16 = jnp.full((16,), 0.5, jnp.float32)

        def zrow(i, carry):
            for j in range(DH // 16):
                rows[0, 0, i, pl.ds(j * 16, 16)] = zero16
            return carry
        lax.fori_loop(0, CH, zrow, 0)

        def zdeg(i, carry):
            dzero[i, pl.ds(0, 16)] = zero16
            return carry
        lax.fori_loop(0, CH, zdeg, 0)

        def orow(i, carry):
            halves[i, pl.ds(0, 16)] = half16
            return carry
        lax.fori_loop(0, CH, orow, 0)

        row0 = sid * ROWS_PER_TILE
        for kblk in range(ROWS_PER_TILE // CH):
            pltpu.sync_copy(rows.at[0, 0],
                            acc.at[pl.ds(row0 + kblk * CH, CH), :])
            pltpu.sync_copy(dzero, dacc.at[pl.ds(row0 + kblk * CH, CH), :])
        plsc.subcore_barrier()

        start = sid * BASE_CHUNKS + jnp.minimum(sid, EXTRA)
        sbias = cid * N

        def fire_idx(g, b):
            base = start + g * SS
            pltpu.async_copy(src_hbm.at[pl.ds(base, SS), :], sidx.at[b],
                             isem[b])
            pltpu.async_copy(dst_hbm.at[pl.ds(base, SS), :], didx.at[b],
                             isem[b])

        def fire_gathers(g, b):
            base = start + g * SS
            pltpu.make_async_copy(src_hbm.at[pl.ds(base, SS), :], sidx.at[b],
                                  isem[b]).wait()
            pltpu.make_async_copy(dst_hbm.at[pl.ds(base, SS), :], didx.at[b],
                                  isem[b]).wait()
            for r in range(SS):
                for j in range(CH // 16):
                    sidx[b, r, pl.ds(j * 16, 16)] = (
                        sidx[b, r, pl.ds(j * 16, 16)] + sbias)
            for r in range(SS):
                pltpu.async_copy(x_hbm.at[sidx.at[b, r]], rows.at[b, r],
                                 gsem[b])

        def drain_gathers(b):
            for r in range(SS):
                pltpu.make_async_copy(x_hbm.at[sidx.at[b, r]],
                                      rows.at[b, r], gsem[b]).wait()

        def fire_scatters(b):
            for r in range(SS):
                pltpu.async_copy(rows.at[b, r], acc.at[didx.at[b, r]],
                                 ssem[b], add=True)
                pltpu.async_copy(halves, dacc.at[didx.at[b, r]],
                                 ssem[b], add=True)

        def drain_scatters(b):
            for r in range(SS):
                pltpu.make_async_copy(rows.at[b, r], acc.at[didx.at[b, r]],
                                      ssem[b]).wait()
                pltpu.make_async_copy(halves, dacc.at[didx.at[b, r]],
                                      ssem[b]).wait()

        fire_idx(0, 0)
        fire_idx(1, 1)
        fire_gathers(0, 0)

        def pair(i, carry):
            fire_gathers(2 * i + 1, 1)
            drain_gathers(0)
            fire_scatters(0)
            drain_scatters(0)

            @pl.when(i < NPAIR - 1)
            def _ia():
                fire_idx(2 * i + 2, 0)
            drain_gathers(1)
            fire_scatters(1)

            @pl.when(i < NPAIR - 1)
            def _ga():
                fire_gathers(2 * i + 2, 0)
            drain_scatters(1)

            @pl.when(i < NPAIR - 1)
            def _ib():
                fire_idx(2 * i + 3, 1)
            return carry
        lax.fori_loop(0, NPAIR, pair, 0)

        @pl.when(sid < EXTRA)
        def _tail():
            base = start + BASE_CHUNKS
            pltpu.sync_copy(src_hbm.at[pl.ds(base, 1), :],
                            sidx.at[0, pl.ds(0, 1)])
            pltpu.sync_copy(dst_hbm.at[pl.ds(base, 1), :],
                            didx.at[0, pl.ds(0, 1)])
            for j in range(CH // 16):
                sidx[0, 0, pl.ds(j * 16, 16)] = (
                    sidx[0, 0, pl.ds(j * 16, 16)] + sbias)
            pltpu.async_copy(x_hbm.at[sidx.at[0, 0]], rows.at[0, 0],
                             gsem[0]).wait()
            pltpu.sync_copy(rows.at[0, 0], acc.at[didx.at[0, 0]], add=True)
            pltpu.sync_copy(halves, dacc.at[didx.at[0, 0]], add=True)

        plsc.subcore_barrier()

        pltpu.sync_copy(acc.at[pl.ds(row0, ROWS_PER_TILE), :],
                        agg_out.at[cid, pl.ds(row0, ROWS_PER_TILE), :])
        pltpu.sync_copy(dacc.at[pl.ds(row0, ROWS_PER_TILE), :],
                        deg_out.at[cid, pl.ds(row0, ROWS_PER_TILE), :])

    return body(xcat, src, dst)


def _phase_a_body(agg_ref, deg_ref, w_ref, cid_ref, z_ref, mu_ref, sums, cnts):
    i = pl.program_id(0)

    @pl.when(i == 0)
    def _init():
        sums[...] = jnp.zeros_like(sums)
        cnts[...] = jnp.zeros_like(cnts)

    h = (lax.dot_general(agg_ref[0], w_ref[0:DH, :], (((1,), (0,)), ((), ())),
                         preferred_element_type=jnp.float32)
         + lax.dot_general(agg_ref[1], w_ref[DH:D, :], (((1,), (0,)), ((), ())),
                           preferred_element_type=jnp.float32))
    dg = deg_ref[0, :, 0:1] + deg_ref[1, :, 0:1]
    h = h / jnp.maximum(dg, 1.0)
    h = jnp.maximum(h, 0.0)
    nrm = jnp.sqrt(jnp.sum(h * h, axis=1, keepdims=True))
    z = h / jnp.maximum(nrm, 1e-12)
    z_ref[...] = z

    cid = cid_ref[0]
    oht = (cid == lax.broadcasted_iota(jnp.int32, (K, 1), 0)
           ).astype(jnp.float32)
    sums[...] += lax.dot_general(oht, z, (((1,), (0,)), ((), ())),
                                 preferred_element_type=jnp.float32)
    cnts[...] += jnp.sum(oht, axis=1, keepdims=True)

    @pl.when(i == NB - 1)
    def _fin():
        mu_ref[...] = sums[...] / jnp.maximum(cnts[...], 1.0)


def _phase_a(agg, deg2, w, cids2):
    return pl.pallas_call(
        _phase_a_body,
        grid=(NB,),
        in_specs=[
            pl.BlockSpec((NC, BN, DH), lambda i: (0, i, 0)),
            pl.BlockSpec((NC, BN, DEGW), lambda i: (0, i, 0)),
            pl.BlockSpec((D, D), lambda i: (0, 0)),
            pl.BlockSpec((1, 1, BN), lambda i: (i, 0, 0)),
        ],
        out_specs=[
            pl.BlockSpec((BN, D), lambda i: (i, 0)),
            pl.BlockSpec((K, D), lambda i: (0, 0)),
        ],
        out_shape=[
            jax.ShapeDtypeStruct((N, D), jnp.float32),
            jax.ShapeDtypeStruct((K, D), jnp.float32),
        ],
        scratch_shapes=[
            pltpu.VMEM((K, D), jnp.float32),
            pltpu.VMEM((K, 1), jnp.float32),
        ],
    )(agg, deg2, w, cids2)


def _phase_b_body(z_ref, mu_ref, dist_ref, r_ref):
    z = z_ref[...]
    mu = mu_ref[...]
    d = lax.dot_general(z, mu, (((1,), (1,)), ((), ())),
                        preferred_element_type=jnp.float32)
    dist_ref[...] = d
    t = TEMP * d
    m = jnp.max(t, axis=1, keepdims=True)
    e = jnp.exp(t - m)
    r_ref[...] = e / jnp.sum(e, axis=1, keepdims=True)


def _phase_b(z_pad, mu):
    return pl.pallas_call(
        _phase_b_body,
        grid=(NB,),
        in_specs=[
            pl.BlockSpec((BN, D), lambda i: (i, 0)),
            pl.BlockSpec((K, D), lambda i: (0, 0)),
        ],
        out_specs=[
            pl.BlockSpec((BN, K), lambda i: (i, 0)),
            pl.BlockSpec((BN, K), lambda i: (i, 0)),
        ],
        out_shape=[
            jax.ShapeDtypeStruct((N, K), jnp.float32),
            jax.ShapeDtypeStruct((N, K), jnp.float32),
        ],
    )(z_pad, mu)


def kernel(x, W_enc, edge_index, community_ids):
    src2 = edge_index[0].reshape(NCHUNKS, CH)
    dst2 = edge_index[1].reshape(NCHUNKS, CH)
    xcat = jnp.concatenate([x[:, :DH], x[:, DH:]], axis=0)
    agg2, deg2 = _sc_edge_agg(xcat, src2, dst2)
    cids2 = community_ids.reshape(NB, 1, BN)
    z, mu = _phase_a(agg2, deg2, W_enc, cids2)
    dist, r = _phase_b(z, mu)
    return (z, mu, r, dist)


# speedup vs baseline: 1.3047x; 1.0229x over previous
import functools

import jax
import jax.numpy as jnp
from jax import lax
from jax.experimental import pallas as pl
from jax.experimental.pallas import tpu as pltpu
from jax.experimental.pallas import tpu_sc as plsc

N = 10000
E = 320000
D = 128
K = 64
TEMP = 30.0

NC = 2
NS = 16
NW = NC * NS
NPAD = 10240
ROWS_PER_TILE = NPAD // NS

CH = 128
NCHUNKS = E // CH
BASE_CHUNKS = NCHUNKS // NS
EXTRA = NCHUNKS - BASE_CHUNKS * NS
SS = 3
NSS = BASE_CHUNKS // SS
NPAIR = NSS // 2

DEGW = 16
DH = D // NC

BN = 1000
NB = N // BN


def _sc_edge_agg(xlr, src2, dst2):
    mesh = plsc.VectorSubcoreMesh(core_axis_name="c", subcore_axis_name="s")

    @functools.partial(
        pl.kernel,
        mesh=mesh,
        out_type=[
            jax.ShapeDtypeStruct((NC, NPAD, DH), jnp.float32),
            jax.ShapeDtypeStruct((NC, NPAD, DEGW), jnp.float32),
        ],
        scratch_types=[
            pltpu.VMEM((2, SS, CH), jnp.int32),
            pltpu.VMEM((2, SS, CH), jnp.int32),
            pltpu.VMEM((2, SS, CH, DH), jnp.float32),
            pltpu.VMEM((CH, DEGW), jnp.float32),
            pltpu.VMEM((CH, DEGW), jnp.float32),
            pltpu.VMEM_SHARED((NPAD, DH), jnp.float32),
            pltpu.VMEM_SHARED((NPAD, DEGW), jnp.float32),
            pltpu.SemaphoreType.DMA,
            pltpu.SemaphoreType.DMA,
            pltpu.SemaphoreType.DMA,
            pltpu.SemaphoreType.DMA,
            pltpu.SemaphoreType.DMA,
            pltpu.SemaphoreType.DMA,
        ],
        compiler_params=pltpu.CompilerParams(use_tc_tiling_on_sc=False),
    )
    def body(xlr_hbm, src_hbm, dst_hbm, agg_out, deg_out,
             sidx, didx, rows, halves, dzero, acc, dacc,
             gsem0, gsem1, ssem0, ssem1, isem0, isem1):
        gsem = (gsem0, gsem1)
        ssem = (ssem0, ssem1)
        isem = (isem0, isem1)
        cid = lax.axis_index("c")
        sid = lax.axis_index("s")
        zero16 = jnp.zeros((16,), jnp.float32)
        half16 = jnp.full((16,), 0.5, jnp.float32)

        def zrow(i, carry):
            for j in range(DH // 16):
                rows[0, 0, i, pl.ds(j * 16, 16)] = zero16
            return carry
        lax.fori_loop(0, CH, zrow, 0)

        def zdeg(i, carry):
            dzero[i, pl.ds(0, 16)] = zero16
            return carry
        lax.fori_loop(0, CH, zdeg, 0)

        def orow(i, carry):
            halves[i, pl.ds(0, 16)] = half16
            return carry
        lax.fori_loop(0, CH, orow, 0)

        row0 = sid * ROWS_PER_TILE
        for kblk in range(ROWS_PER_TILE // CH):
            pltpu.sync_copy(rows.at[0, 0],
                            acc.at[pl.ds(row0 + kblk * CH, CH), :])
            pltpu.sync_copy(dzero, dacc.at[pl.ds(row0 + kblk * CH, CH), :])
        plsc.subcore_barrier()

        start = sid * BASE_CHUNKS + jnp.minimum(sid, EXTRA)

        def fire_idx(g, b):
            base = start + g * SS
            pltpu.async_copy(src_hbm.at[pl.ds(base, SS), :], sidx.at[b],
                             isem[b])
            pltpu.async_copy(dst_hbm.at[pl.ds(base, SS), :], didx.at[b],
                             isem[b])

        def fire_gathers(g, b):
            base = start + g * SS
            pltpu.make_async_copy(src_hbm.at[pl.ds(base, SS), :], sidx.at[b],
                                  isem[b]).wait()
            pltpu.make_async_copy(dst_hbm.at[pl.ds(base, SS), :], didx.at[b],
                                  isem[b]).wait()

            for r in range(SS):
                pltpu.async_copy(xlr_hbm.at[cid].at[sidx.at[b, r]],
                                 rows.at[b, r], gsem[b])

        def drain_gathers(b):
            for r in range(SS):
                pltpu.make_async_copy(xlr_hbm.at[cid].at[sidx.at[b, r]],
                                      rows.at[b, r], gsem[b]).wait()

        def fire_scatters(b):
            for r in range(SS):
                pltpu.async_copy(rows.at[b, r], acc.at[didx.at[b, r]],
                                 ssem[b], add=True)
                pltpu.async_copy(halves, dacc.at[didx.at[b, r]],
                                 ssem[b], add=True)

        def drain_scatters(b):
            for r in range(SS):
                pltpu.make_async_copy(rows.at[b, r], acc.at[didx.at[b, r]],
                                      ssem[b]).wait()
                pltpu.make_async_copy(halves, dacc.at[didx.at[b, r]],
                                      ssem[b]).wait()

        fire_idx(0, 0)
        fire_idx(1, 1)
        fire_gathers(0, 0)

        def pair(i, carry):
            fire_gathers(2 * i + 1, 1)
            drain_gathers(0)
            fire_scatters(0)
            drain_scatters(0)

            @pl.when(i < NPAIR - 1)
            def _ia():
                fire_idx(2 * i + 2, 0)
            drain_gathers(1)
            fire_scatters(1)

            @pl.when(i < NPAIR - 1)
            def _ga():
                fire_gathers(2 * i + 2, 0)
            drain_scatters(1)

            @pl.when(i < NPAIR - 1)
            def _ib():
                fire_idx(2 * i + 3, 1)
            return carry
        lax.fori_loop(0, NPAIR, pair, 0)

        @pl.when(sid < EXTRA)
        def _tail():
            base = start + BASE_CHUNKS
            pltpu.sync_copy(src_hbm.at[pl.ds(base, 1), :],
                            sidx.at[0, pl.ds(0, 1)])
            pltpu.sync_copy(dst_hbm.at[pl.ds(base, 1), :],
                            didx.at[0, pl.ds(0, 1)])

            pltpu.async_copy(xlr_hbm.at[cid].at[sidx.at[0, 0]],
                             rows.at[0, 0], gsem[0]).wait()
            pltpu.sync_copy(rows.at[0, 0], acc.at[didx.at[0, 0]], add=True)
            pltpu.sync_copy(halves, dacc.at[didx.at[0, 0]], add=True)

        plsc.subcore_barrier()

        pltpu.sync_copy(acc.at[pl.ds(row0, ROWS_PER_TILE), :],
                        agg_out.at[cid, pl.ds(row0, ROWS_PER_TILE), :])
        pltpu.sync_copy(dacc.at[pl.ds(row0, ROWS_PER_TILE), :],
                        deg_out.at[cid, pl.ds(row0, ROWS_PER_TILE), :])

    return body(xlr, src2, dst2)


def _phase_a_body(agg_ref, deg_ref, w_ref, cid_ref, z_ref, mu_ref, sums, cnts):
    i = pl.program_id(0)

    @pl.when(i == 0)
    def _init():
        sums[...] = jnp.zeros_like(sums)
        cnts[...] = jnp.zeros_like(cnts)

    h = (lax.dot_general(agg_ref[0], w_ref[0:DH, :], (((1,), (0,)), ((), ())),
                         preferred_element_type=jnp.float32)
         + lax.dot_general(agg_ref[1], w_ref[DH:D, :], (((1,), (0,)), ((), ())),
                           preferred_element_type=jnp.float32))
    dg = deg_ref[0, :, 0:1] + deg_ref[1, :, 0:1]
    h = h / jnp.maximum(dg, 1.0)
    h = jnp.maximum(h, 0.0)
    nrm = jnp.sqrt(jnp.sum(h * h, axis=1, keepdims=True))
    z = h / jnp.maximum(nrm, 1e-12)
    z_ref[...] = z

    cid = cid_ref[0]
    oht = (cid == lax.broadcasted_iota(jnp.int32, (K, 1), 0)
           ).astype(jnp.float32)
    sums[...] += lax.dot_general(oht, z, (((1,), (0,)), ((), ())),
                                 preferred_element_type=jnp.float32)
    cnts[...] += jnp.sum(oht, axis=1, keepdims=True)

    @pl.when(i == NB - 1)
    def _fin():
        mu_ref[...] = sums[...] / jnp.maximum(cnts[...], 1.0)


def _phase_a(agg, deg2, w, cids2):
    return pl.pallas_call(
        _phase_a_body,
        grid=(NB,),
        in_specs=[
            pl.BlockSpec((NC, BN, DH), lambda i: (0, i, 0)),
            pl.BlockSpec((NC, BN, DEGW), lambda i: (0, i, 0)),
            pl.BlockSpec((D, D), lambda i: (0, 0)),
            pl.BlockSpec((1, 1, BN), lambda i: (i, 0, 0)),
        ],
        out_specs=[
            pl.BlockSpec((BN, D), lambda i: (i, 0)),
            pl.BlockSpec((K, D), lambda i: (0, 0)),
        ],
        out_shape=[
            jax.ShapeDtypeStruct((N, D), jnp.float32),
            jax.ShapeDtypeStruct((K, D), jnp.float32),
        ],
        scratch_shapes=[
            pltpu.VMEM((K, D), jnp.float32),
            pltpu.VMEM((K, 1), jnp.float32),
        ],
    )(agg, deg2, w, cids2)


def _phase_b_body(z_ref, mu_ref, dist_ref, r_ref):
    z = z_ref[...]
    mu = mu_ref[...]
    d = lax.dot_general(z, mu, (((1,), (1,)), ((), ())),
                        preferred_element_type=jnp.float32)
    dist_ref[...] = d
    t = TEMP * d
    m = jnp.max(t, axis=1, keepdims=True)
    e = jnp.exp(t - m)
    r_ref[...] = e / jnp.sum(e, axis=1, keepdims=True)


def _phase_b(z_pad, mu):
    return pl.pallas_call(
        _phase_b_body,
        grid=(NB,),
        in_specs=[
            pl.BlockSpec((BN, D), lambda i: (i, 0)),
            pl.BlockSpec((K, D), lambda i: (0, 0)),
        ],
        out_specs=[
            pl.BlockSpec((BN, K), lambda i: (i, 0)),
            pl.BlockSpec((BN, K), lambda i: (i, 0)),
        ],
        out_shape=[
            jax.ShapeDtypeStruct((N, K), jnp.float32),
            jax.ShapeDtypeStruct((N, K), jnp.float32),
        ],
    )(z_pad, mu)


def kernel(x, W_enc, edge_index, community_ids):
    src2 = edge_index[0].reshape(NCHUNKS, CH)
    dst2 = edge_index[1].reshape(NCHUNKS, CH)
    xlr = jnp.stack([x[:, :DH], x[:, DH:]])
    agg2, deg2 = _sc_edge_agg(xlr, src2, dst2)
    cids2 = community_ids.reshape(NB, 1, BN)
    z, mu = _phase_a(agg2, deg2, W_enc, cids2)
    dist, r = _phase_b(z, mu)
    return (z, mu, r, dist)
